# baseline (speedup 1.0000x reference)
import jax
import jax.numpy as jnp
from jax.experimental import pallas as pl
from jax.experimental.pallas import tpu as pltpu

N = 256
D = 256
H = 512
NREL = 3
NLAYERS = 2

BI = 8


def _probs_kernel(xT_ref, xi_ref, w0aT_ref, w0bT_ref, w0cT_ref, b0_ref,
                  w1T_ref, b1_ref, woutT_ref, bout_ref, out_ref,
                  xaT_ref, w0c_bf_ref, w1_bf_ref, xT_bf_ref):
    i = pl.program_id(0)

    @pl.when(i == 0)
    def _():
        xaT_ref[...] = jnp.dot(w0aT_ref[...], xT_ref[...],
                               preferred_element_type=jnp.float32)
        w0c_bf_ref[...] = w0cT_ref[...].astype(jnp.bfloat16)
        w1_bf_ref[...] = w1T_ref[...].astype(jnp.bfloat16)
        xT_bf_ref[...] = xT_ref[...].astype(jnp.bfloat16)

    xi_cols = xi_ref[...].T
    xbb = jnp.dot(w0bT_ref[...], xi_cols,
                  preferred_element_type=jnp.float32) + b0_ref[...]
    xi_bf = xi_cols.astype(jnp.bfloat16)
    xT_bf = xT_bf_ref[...]
    xaT = xaT_ref[...]
    w0c_bf = w0c_bf_ref[...]
    w1_bf = w1_bf_ref[...]
    wout_bf = woutT_ref[...].astype(jnp.bfloat16)
    b1 = b1_ref[...]
    s_blocks = []
    for a in range(BI):
        p_a = xT_bf * xi_bf[:, a:a + 1]
        hT_a = jnp.dot(w0c_bf, p_a, preferred_element_type=jnp.float32)
        g_a = jnp.maximum(hT_a + (xaT + xbb[:, a:a + 1]),
                          0.0).astype(jnp.bfloat16)
        h2_a = jnp.maximum(jnp.dot(w1_bf, g_a,
                                   preferred_element_type=jnp.float32) + b1,
                           0.0).astype(jnp.bfloat16)
        s_blocks.append(jnp.dot(wout_bf, h2_a,
                                preferred_element_type=jnp.float32))
    s = jnp.concatenate(s_blocks, axis=1) + bout_ref[...]
    s0, s1, s2 = s[0:1, :], s[1:2, :], s[2:3, :]
    m = jnp.maximum(jnp.maximum(s0, s1), s2)
    e0 = jnp.exp(s0 - m)
    e1 = jnp.exp(s1 - m)
    e2 = jnp.exp(s2 - m)
    inv = 1.0 / (e0 + e1 + e2)
    c = jax.lax.broadcasted_iota(jnp.int32, (1, BI * N), 1)
    keep = (c & (N - 1)) != i * BI + (c >> 8)
    zero = jnp.zeros((1, BI * N), jnp.float32)
    out_ref[0:1, :] = jnp.where(keep, e0 * inv, zero)
    out_ref[1:2, :] = jnp.where(keep, e1 * inv, zero)
    out_ref[2:3, :] = jnp.where(keep, e2 * inv, zero)


def _bigcn_kernel(probs_ref, x_ref, fww_ref, fwb_ref, bww_ref, bwb_ref,
                  l1w_ref, l1b_ref, out_ref):
    out = x_ref[...]
    for l in range(NLAYERS):
        rel_sum = jnp.zeros((N, D), dtype=jnp.float32)
        for r in range(NREL):
            a = probs_ref[r]
            fw = jnp.dot(a, jnp.dot(out, fww_ref[l, r],
                                    preferred_element_type=jnp.float32),
                         preferred_element_type=jnp.float32) + fwb_ref[l, r][None, :]
            bw = jnp.dot(a.T, jnp.dot(out, bww_ref[l, r],
                                      preferred_element_type=jnp.float32),
                         preferred_element_type=jnp.float32) + bwb_ref[l, r][None, :]
            rel_sum = rel_sum + jnp.concatenate([bw, fw], axis=-1)
        out = jnp.dot(jax.nn.relu(rel_sum), l1w_ref[l],
                      preferred_element_type=jnp.float32) + l1b_ref[l][None, :] + out
    out_ref[...] = out


def kernel(candidate_embs, ffnn_W0, ffnn_b0, ffnn_W1, ffnn_b1, ffnn_Wout,
           ffnn_bout, gcn_fw_W, gcn_fw_b, gcn_bw_W, gcn_bw_b, lin1_W, lin1_b):
    xT = candidate_embs.T
    w0aT = ffnn_W0[:D].T
    w0bT = ffnn_W0[D:2 * D].T
    w0cT = ffnn_W0[2 * D:].T
    grid = N // BI
    full = lambda *shape: pl.BlockSpec(shape, lambda i: (0,) * len(shape))
    probs2d = pl.pallas_call(
        _probs_kernel,
        grid=(grid,),
        in_specs=[
            full(D, N),
            pl.BlockSpec((BI, D), lambda i: (i, 0)),
            full(H, D),
            full(H, D),
            full(H, D),
            full(H, 1),
            full(H, H),
            full(H, 1),
            full(NREL, H),
            full(NREL, 1),
        ],
        out_specs=pl.BlockSpec((NREL, BI * N), lambda i: (0, i)),
        out_shape=jax.ShapeDtypeStruct((NREL, N * N), jnp.float32),
        scratch_shapes=[
            pltpu.VMEM((H, N), jnp.float32),
            pltpu.VMEM((H, D), jnp.bfloat16),
            pltpu.VMEM((H, H), jnp.bfloat16),
            pltpu.VMEM((D, N), jnp.bfloat16),
        ],
    )(xT, candidate_embs, w0aT, w0bT, w0cT, ffnn_b0[:, None], ffnn_W1.T,
      ffnn_b1[:, None], ffnn_Wout.T, ffnn_bout[:, None])
    probs = probs2d.reshape(NREL, N, N)

    out = pl.pallas_call(
        _bigcn_kernel,
        out_shape=jax.ShapeDtypeStruct((N, D), jnp.float32),
    )(probs, candidate_embs, gcn_fw_W, gcn_fw_b, gcn_bw_W, gcn_bw_b,
      lin1_W, lin1_b)
    return out


# speedup vs baseline: 2.2260x; 2.2260x over previous
import jax
import jax.numpy as jnp
from jax.experimental import pallas as pl
from jax.experimental.pallas import tpu as pltpu

N = 256
D = 256
H = 512
NREL = 3
NLAYERS = 2

BI = 32


def _probs_kernel(xT_ref, xi_ref, w0aT_ref, w0bT_ref, w0cT_ref, b0_ref,
                  w1T_ref, b1_ref, woutT_ref, bout_ref, out_ref,
                  xaT_ref, w0c_bf_ref, w1_bf_ref, xT_bf_ref):
    i = pl.program_id(0)

    @pl.when(i == 0)
    def _():
        xaT_ref[...] = jnp.dot(w0aT_ref[...], xT_ref[...],
                               preferred_element_type=jnp.float32)
        w0c_bf_ref[...] = w0cT_ref[...].astype(jnp.bfloat16)
        w1_bf_ref[...] = w1T_ref[...].astype(jnp.bfloat16)
        xT_bf_ref[...] = xT_ref[...].astype(jnp.bfloat16)

    xi_cols = xi_ref[...].T
    xbb = jnp.dot(w0bT_ref[...], xi_cols,
                  preferred_element_type=jnp.float32) + b0_ref[...]
    xi_bf = xi_cols.astype(jnp.bfloat16)
    xT_bf = xT_bf_ref[...]
    pT = jnp.concatenate([xT_bf * xi_bf[:, a:a + 1] for a in range(BI)],
                         axis=1)
    hT = jnp.dot(w0c_bf_ref[...], pT, preferred_element_type=jnp.float32)
    xaT = xaT_ref[...]
    h_bf = jnp.concatenate(
        [jnp.maximum(hT[:, a * N:(a + 1) * N] + (xaT + xbb[:, a:a + 1]),
                     0.0).astype(jnp.bfloat16) for a in range(BI)],
        axis=1)
    h2 = jnp.dot(w1_bf_ref[...], h_bf,
                 preferred_element_type=jnp.float32) + b1_ref[...]
    h2_bf = jnp.maximum(h2, 0.0).astype(jnp.bfloat16)
    s = jnp.dot(woutT_ref[...].astype(jnp.bfloat16), h2_bf,
                preferred_element_type=jnp.float32) + bout_ref[...]
    s0, s1, s2 = s[0:1, :], s[1:2, :], s[2:3, :]
    m = jnp.maximum(jnp.maximum(s0, s1), s2)
    e0 = jnp.exp(s0 - m)
    e1 = jnp.exp(s1 - m)
    e2 = jnp.exp(s2 - m)
    inv = 1.0 / (e0 + e1 + e2)
    c = jax.lax.broadcasted_iota(jnp.int32, (1, BI * N), 1)
    keep = (c & (N - 1)) != i * BI + (c >> 8)
    zero = jnp.zeros((1, BI * N), jnp.float32)
    out_ref[0:1, :] = jnp.where(keep, e0 * inv, zero)
    out_ref[1:2, :] = jnp.where(keep, e1 * inv, zero)
    out_ref[2:3, :] = jnp.where(keep, e2 * inv, zero)


def _bigcn_kernel(probs_ref, x_ref, fww_ref, fwb_ref, bww_ref, bwb_ref,
                  l1w_ref, l1b_ref, out_ref):
    out = x_ref[...]
    for l in range(NLAYERS):
        rel_sum = jnp.zeros((N, D), dtype=jnp.float32)
        for r in range(NREL):
            a = probs_ref[r]
            fw = jnp.dot(a, jnp.dot(out, fww_ref[l, r],
                                    preferred_element_type=jnp.float32),
                         preferred_element_type=jnp.float32) + fwb_ref[l, r][None, :]
            bw = jnp.dot(a.T, jnp.dot(out, bww_ref[l, r],
                                      preferred_element_type=jnp.float32),
                         preferred_element_type=jnp.float32) + bwb_ref[l, r][None, :]
            rel_sum = rel_sum + jnp.concatenate([bw, fw], axis=-1)
        out = jnp.dot(jax.nn.relu(rel_sum), l1w_ref[l],
                      preferred_element_type=jnp.float32) + l1b_ref[l][None, :] + out
    out_ref[...] = out


def kernel(candidate_embs, ffnn_W0, ffnn_b0, ffnn_W1, ffnn_b1, ffnn_Wout,
           ffnn_bout, gcn_fw_W, gcn_fw_b, gcn_bw_W, gcn_bw_b, lin1_W, lin1_b):
    xT = candidate_embs.T
    w0aT = ffnn_W0[:D].T
    w0bT = ffnn_W0[D:2 * D].T
    w0cT = ffnn_W0[2 * D:].T
    grid = N // BI
    full = lambda *shape: pl.BlockSpec(shape, lambda i: (0,) * len(shape))
    probs2d = pl.pallas_call(
        _probs_kernel,
        grid=(grid,),
        in_specs=[
            full(D, N),
            pl.BlockSpec((BI, D), lambda i: (i, 0)),
            full(H, D),
            full(H, D),
            full(H, D),
            full(H, 1),
            full(H, H),
            full(H, 1),
            full(NREL, H),
            full(NREL, 1),
        ],
        out_specs=pl.BlockSpec((NREL, BI * N), lambda i: (0, i)),
        out_shape=jax.ShapeDtypeStruct((NREL, N * N), jnp.float32),
        scratch_shapes=[
            pltpu.VMEM((H, N), jnp.float32),
            pltpu.VMEM((H, D), jnp.bfloat16),
            pltpu.VMEM((H, H), jnp.bfloat16),
            pltpu.VMEM((D, N), jnp.bfloat16),
        ],
    )(xT, candidate_embs, w0aT, w0bT, w0cT, ffnn_b0[:, None], ffnn_W1.T,
      ffnn_b1[:, None], ffnn_Wout.T, ffnn_bout[:, None])
    probs = probs2d.reshape(NREL, N, N)

    out = pl.pallas_call(
        _bigcn_kernel,
        out_shape=jax.ShapeDtypeStruct((N, D), jnp.float32),
    )(probs, candidate_embs, gcn_fw_W, gcn_fw_b, gcn_bw_W, gcn_bw_b,
      lin1_W, lin1_b)
    return out


# speedup vs baseline: 2.4038x; 1.0799x over previous
import jax
import jax.numpy as jnp
from jax.experimental import pallas as pl
from jax.experimental.pallas import tpu as pltpu

N = 256
D = 256
H = 512
NREL = 3
NLAYERS = 2

BI = 64


def _probs_kernel(xT_ref, xi_ref, w0aT_ref, w0bT_ref, w0cT_ref, b0_ref,
                  w1T_ref, b1_ref, woutT_ref, bout_ref, out_ref, xaT_ref):
    i = pl.program_id(0)

    @pl.when(i == 0)
    def _():
        xaT_ref[...] = jnp.dot(w0aT_ref[...], xT_ref[...],
                               preferred_element_type=jnp.float32).astype(
                                   jnp.bfloat16)

    xi_bf = xi_ref[...].T
    xbb = (jnp.dot(w0bT_ref[...], xi_bf,
                   preferred_element_type=jnp.float32)
           + b0_ref[...]).astype(jnp.bfloat16)
    xT_bf = xT_ref[...]
    pT = jnp.concatenate([xT_bf * xi_bf[:, a:a + 1] for a in range(BI)],
                         axis=1)
    hT = jnp.dot(w0cT_ref[...], pT,
                 preferred_element_type=jnp.float32).astype(jnp.bfloat16)
    xaT = xaT_ref[...]
    zero_bf = jnp.zeros((), jnp.bfloat16)
    h_bf = jnp.concatenate(
        [jnp.maximum(hT[:, a * N:(a + 1) * N] + (xaT + xbb[:, a:a + 1]),
                     zero_bf) for a in range(BI)],
        axis=1)
    h2 = jnp.dot(w1T_ref[...], h_bf,
                 preferred_element_type=jnp.float32).astype(jnp.bfloat16)
    h2_bf = jnp.maximum(h2 + b1_ref[...].astype(jnp.bfloat16), zero_bf)
    s = jnp.dot(woutT_ref[...], h2_bf,
                preferred_element_type=jnp.float32) + bout_ref[...]
    s0, s1, s2 = s[0:1, :], s[1:2, :], s[2:3, :]
    m = jnp.maximum(jnp.maximum(s0, s1), s2)
    e0 = jnp.exp(s0 - m)
    e1 = jnp.exp(s1 - m)
    e2 = jnp.exp(s2 - m)
    inv = 1.0 / (e0 + e1 + e2)
    c = jax.lax.broadcasted_iota(jnp.int32, (1, BI * N), 1)
    keep = (c & (N - 1)) != i * BI + (c >> 8)
    zero = jnp.zeros((1, BI * N), jnp.float32)
    out_ref[0:1, :] = jnp.where(keep, e0 * inv, zero).astype(jnp.bfloat16)
    out_ref[1:2, :] = jnp.where(keep, e1 * inv, zero).astype(jnp.bfloat16)
    out_ref[2:3, :] = jnp.where(keep, e2 * inv, zero).astype(jnp.bfloat16)


def _bigcn_kernel(probs_ref, x_ref, fww_ref, fwb_ref, bww_ref, bwb_ref,
                  l1w_ref, l1b_ref, out_ref):
    out = x_ref[...]
    a_all = [probs_ref[r].astype(jnp.float32) for r in range(NREL)]
    at_all = [a.T for a in a_all]
    for l in range(NLAYERS):
        rel_sum = jnp.zeros((N, D), dtype=jnp.float32)
        for r in range(NREL):
            fw = jnp.dot(a_all[r], jnp.dot(out, fww_ref[l, r],
                                           preferred_element_type=jnp.float32),
                         preferred_element_type=jnp.float32) + fwb_ref[l, r][None, :]
            bw = jnp.dot(at_all[r], jnp.dot(out, bww_ref[l, r],
                                            preferred_element_type=jnp.float32),
                         preferred_element_type=jnp.float32) + bwb_ref[l, r][None, :]
            rel_sum = rel_sum + jnp.concatenate([bw, fw], axis=-1)
        out = jnp.dot(jax.nn.relu(rel_sum), l1w_ref[l],
                      preferred_element_type=jnp.float32) + l1b_ref[l][None, :] + out
    out_ref[...] = out


def kernel(candidate_embs, ffnn_W0, ffnn_b0, ffnn_W1, ffnn_b1, ffnn_Wout,
           ffnn_bout, gcn_fw_W, gcn_fw_b, gcn_bw_W, gcn_bw_b, lin1_W, lin1_b):
    x_bf = candidate_embs.astype(jnp.bfloat16)
    xT = x_bf.T
    w0aT = ffnn_W0[:D].T.astype(jnp.bfloat16)
    w0bT = ffnn_W0[D:2 * D].T.astype(jnp.bfloat16)
    w0cT = ffnn_W0[2 * D:].T.astype(jnp.bfloat16)
    grid = N // BI
    full = lambda *shape: pl.BlockSpec(shape, lambda i: (0,) * len(shape))
    probs2d = pl.pallas_call(
        _probs_kernel,
        grid=(grid,),
        in_specs=[
            full(D, N),
            pl.BlockSpec((BI, D), lambda i: (i, 0)),
            full(H, D),
            full(H, D),
            full(H, D),
            full(H, 1),
            full(H, H),
            full(H, 1),
            full(NREL, H),
            full(NREL, 1),
        ],
        out_specs=pl.BlockSpec((NREL, BI * N), lambda i: (0, i)),
        out_shape=jax.ShapeDtypeStruct((NREL, N * N), jnp.bfloat16),
        scratch_shapes=[
            pltpu.VMEM((H, N), jnp.bfloat16),
        ],
    )(xT, x_bf, w0aT, w0bT, w0cT, ffnn_b0[:, None],
      ffnn_W1.T.astype(jnp.bfloat16), ffnn_b1[:, None],
      ffnn_Wout.T.astype(jnp.bfloat16), ffnn_bout[:, None])
    probs = probs2d.reshape(NREL, N, N)

    out = pl.pallas_call(
        _bigcn_kernel,
        out_shape=jax.ShapeDtypeStruct((N, D), jnp.float32),
    )(probs, candidate_embs, gcn_fw_W, gcn_fw_b, gcn_bw_W, gcn_bw_b,
      lin1_W, lin1_b)
    return out


# speedup vs baseline: 2.4227x; 1.0079x over previous
import jax
import jax.numpy as jnp
from jax.experimental import pallas as pl
from jax.experimental.pallas import tpu as pltpu

N = 256
D = 256
H = 512
NREL = 3
NLAYERS = 2

BI = 64


def _probs_kernel(xT_ref, xi_ref, w0aT_ref, w0bT_ref, w0cT_ref, b0_ref,
                  w1T_ref, b1_ref, woutT_ref, bout_ref, out_ref, xaT_ref):
    i = pl.program_id(0)

    @pl.when(i == 0)
    def _():
        xaT_ref[...] = jnp.dot(w0aT_ref[...], xT_ref[...],
                               preferred_element_type=jnp.float32).astype(
                                   jnp.bfloat16)

    xi_bf = xi_ref[0]
    xbb = (jnp.dot(w0bT_ref[...], xi_bf,
                   preferred_element_type=jnp.float32)
           + b0_ref[...]).astype(jnp.bfloat16)
    xT_bf = xT_ref[...]
    pT = jnp.concatenate([xT_bf * xi_bf[:, a:a + 1] for a in range(BI)],
                         axis=1)
    hT = jnp.dot(w0cT_ref[...], pT,
                 preferred_element_type=jnp.float32).astype(jnp.bfloat16)
    xaT = xaT_ref[...]
    zero_bf = jnp.zeros((), jnp.bfloat16)
    h_bf = jnp.concatenate(
        [jnp.maximum(hT[:, a * N:(a + 1) * N] + (xaT + xbb[:, a:a + 1]),
                     zero_bf) for a in range(BI)],
        axis=1)
    h2 = jnp.dot(w1T_ref[...], h_bf,
                 preferred_element_type=jnp.float32).astype(jnp.bfloat16)
    h2_bf = jnp.maximum(h2 + b1_ref[...].astype(jnp.bfloat16), zero_bf)
    s = jnp.dot(woutT_ref[...], h2_bf,
                preferred_element_type=jnp.float32) + bout_ref[...]
    s0, s1, s2 = s[0:1, :], s[1:2, :], s[2:3, :]
    m = jnp.maximum(jnp.maximum(s0, s1), s2)
    e0 = jnp.exp(s0 - m)
    e1 = jnp.exp(s1 - m)
    e2 = jnp.exp(s2 - m)
    inv = 1.0 / (e0 + e1 + e2)
    c = jax.lax.broadcasted_iota(jnp.int32, (1, BI * N), 1)
    keep = (c & (N - 1)) != i * BI + (c >> 8)
    zero = jnp.zeros((1, BI * N), jnp.float32)
    out_ref[0:1, :] = jnp.where(keep, e0 * inv, zero).astype(jnp.bfloat16)
    out_ref[1:2, :] = jnp.where(keep, e1 * inv, zero).astype(jnp.bfloat16)
    out_ref[2:3, :] = jnp.where(keep, e2 * inv, zero).astype(jnp.bfloat16)


def _bigcn_kernel(probs_ref, x_ref, fww_ref, fwb_ref, bww_ref, bwb_ref,
                  l1w_ref, l1b_ref, out_ref):
    out = x_ref[...]
    a_all = [probs_ref[r].astype(jnp.float32) for r in range(NREL)]
    at_all = [a.T for a in a_all]
    for l in range(NLAYERS):
        rel_sum = jnp.zeros((N, D), dtype=jnp.float32)
        for r in range(NREL):
            fw = jnp.dot(a_all[r], jnp.dot(out, fww_ref[l, r],
                                           preferred_element_type=jnp.float32),
                         preferred_element_type=jnp.float32) + fwb_ref[l, r][None, :]
            bw = jnp.dot(at_all[r], jnp.dot(out, bww_ref[l, r],
                                            preferred_element_type=jnp.float32),
                         preferred_element_type=jnp.float32) + bwb_ref[l, r][None, :]
            rel_sum = rel_sum + jnp.concatenate([bw, fw], axis=-1)
        out = jnp.dot(jax.nn.relu(rel_sum), l1w_ref[l],
                      preferred_element_type=jnp.float32) + l1b_ref[l][None, :] + out
    out_ref[...] = out


def kernel(candidate_embs, ffnn_W0, ffnn_b0, ffnn_W1, ffnn_b1, ffnn_Wout,
           ffnn_bout, gcn_fw_W, gcn_fw_b, gcn_bw_W, gcn_bw_b, lin1_W, lin1_b):
    x_bf = candidate_embs.astype(jnp.bfloat16)
    xT = x_bf.T
    w0aT = ffnn_W0[:D].T.astype(jnp.bfloat16)
    w0bT = ffnn_W0[D:2 * D].T.astype(jnp.bfloat16)
    w0cT = ffnn_W0[2 * D:].T.astype(jnp.bfloat16)
    grid = N // BI
    full = lambda *shape: pl.BlockSpec(shape, lambda i: (0,) * len(shape))
    probs2d = pl.pallas_call(
        _probs_kernel,
        grid=(grid,),
        in_specs=[
            full(D, N),
            pl.BlockSpec((1, D, BI), lambda i: (i, 0, 0)),
            full(H, D),
            full(H, D),
            full(H, D),
            full(H, 1),
            full(H, H),
            full(H, 1),
            full(NREL, H),
            full(NREL, 1),
        ],
        out_specs=pl.BlockSpec((NREL, BI * N), lambda i: (0, i)),
        out_shape=jax.ShapeDtypeStruct((NREL, N * N), jnp.bfloat16),
        scratch_shapes=[
            pltpu.VMEM((H, N), jnp.bfloat16),
        ],
    )(xT, x_bf.reshape(grid, BI, D).transpose(0, 2, 1), w0aT, w0bT, w0cT,
      ffnn_b0[:, None],
      ffnn_W1.T.astype(jnp.bfloat16), ffnn_b1[:, None],
      ffnn_Wout.T.astype(jnp.bfloat16), ffnn_bout[:, None])
    probs = probs2d.reshape(NREL, N, N)

    out = pl.pallas_call(
        _bigcn_kernel,
        out_shape=jax.ShapeDtypeStruct((N, D), jnp.float32),
    )(probs, candidate_embs, gcn_fw_W, gcn_fw_b, gcn_bw_W, gcn_bw_b,
      lin1_W, lin1_b)
    return out
